# baseline (speedup 1.0000x reference)
import jax
import jax.numpy as jnp
from jax.experimental import pallas as pl

_BM = 256


def _nm24_body(x_ref, o_ref):
    n = x_ref.shape[1]
    shape = (x_ref.shape[0], 128)
    lane = jax.lax.broadcasted_iota(jnp.uint32, shape, 1)
    p = lane & 3
    tie = (p < 2).astype(jnp.uint32) | jnp.uint32(0x80000000)
    perms = [((lane & ~jnp.uint32(3)) | ((lane + e) & 3)).astype(jnp.int32)
             for e in (1, 2, 3)]

    for c in range(n // 128):
        x = x_ref[:, c * 128:(c + 1) * 128]
        bits = jax.lax.bitcast_convert_type(x, jnp.uint32)
        key = jax.lax.bitcast_convert_type((bits << 1) ^ tie, jnp.int32)
        m1 = jnp.take_along_axis(key, perms[0], axis=1)
        m2 = jnp.take_along_axis(key, perms[1], axis=1)
        m3 = jnp.take_along_axis(key, perms[2], axis=1)
        med = jnp.maximum(jnp.minimum(m1, m2),
                          jnp.minimum(jnp.maximum(m1, m2), m3 + 1))
        o_ref[:, c * 128:(c + 1) * 128] = jnp.where(med > key,
                                                    jnp.zeros_like(x), x)


def _nm24(weights):
    m, n = weights.shape
    grid = (m // _BM,)
    return pl.pallas_call(
        _nm24_body,
        grid=grid,
        in_specs=[pl.BlockSpec((_BM, n), lambda i: (i, 0))],
        out_specs=pl.BlockSpec((_BM, n), lambda i: (i, 0)),
        out_shape=jax.ShapeDtypeStruct((m, n), weights.dtype),
    )(weights)


@jax.jit
def kernel(weights):
    return _nm24(weights)


# speedup vs baseline: 1.0218x; 1.0218x over previous
import jax
import jax.numpy as jnp
from jax.experimental import pallas as pl

_BM = 512


def _nm24_body(x_ref, o_ref):
    n = x_ref.shape[1]
    shape = (x_ref.shape[0], 128)
    lane = jax.lax.broadcasted_iota(jnp.uint32, shape, 1)
    p = lane & 3
    tie = (p < 2).astype(jnp.uint32) | jnp.uint32(0x80000000)
    perms = [((lane & ~jnp.uint32(3)) | ((lane + e) & 3)).astype(jnp.int32)
             for e in (1, 2, 3)]

    for c in range(n // 128):
        x = x_ref[:, c * 128:(c + 1) * 128]
        bits = jax.lax.bitcast_convert_type(x, jnp.uint32)
        key = jax.lax.bitcast_convert_type((bits << 1) ^ tie, jnp.int32)
        m1 = jnp.take_along_axis(key, perms[0], axis=1)
        m2 = jnp.take_along_axis(key, perms[1], axis=1)
        m3 = jnp.take_along_axis(key, perms[2], axis=1)
        med = jnp.maximum(jnp.minimum(m1, m2),
                          jnp.minimum(jnp.maximum(m1, m2), m3 + 1))
        o_ref[:, c * 128:(c + 1) * 128] = jnp.where(med > key,
                                                    jnp.zeros_like(x), x)


def _nm24(weights):
    m, n = weights.shape
    grid = (m // _BM,)
    return pl.pallas_call(
        _nm24_body,
        grid=grid,
        in_specs=[pl.BlockSpec((_BM, n), lambda i: (i, 0))],
        out_specs=pl.BlockSpec((_BM, n), lambda i: (i, 0)),
        out_shape=jax.ShapeDtypeStruct((m, n), weights.dtype),
    )(weights)


@jax.jit
def kernel(weights):
    return _nm24(weights)
